# baseline (speedup 1.0000x reference)
import jax
import jax.numpy as jnp
from jax.experimental import pallas as pl
from jax.experimental.pallas import tpu as pltpu

A = 8


def _body(h_ref, pi_ref, act_ref, obs_ref,
          kW1_ref, kb1_ref, kW2_ref, kb2_ref,
          qW1_ref, qb1_ref, qW2_ref, qb2_ref,
          out_ref, w_ref):
    R, D = h_ref.shape
    NA = pi_ref.shape[1]
    E = R // A

    hb = h_ref[...]
    K = jnp.dot(jnp.tanh(jnp.dot(hb, kW1_ref[...], preferred_element_type=jnp.float32)
                         + kb1_ref[...]),
                kW2_ref[...], preferred_element_type=jnp.float32) + kb2_ref[...]
    Q = jnp.dot(jnp.tanh(jnp.dot(hb, qW1_ref[...], preferred_element_type=jnp.float32)
                         + qb1_ref[...]),
                qW2_ref[...], preferred_element_type=jnp.float32) + qb2_ref[...]

    K3 = K.reshape(E, A, K.shape[-1])
    Q3 = Q.reshape(E, A, K.shape[-1])
    S = jnp.sum(Q3[:, :, None, :] * K3[:, None, :, :], axis=-1)
    w3 = jax.nn.sigmoid(S * 0.125)

    pi3 = pi_ref[...].reshape(E, A, NA)
    act3 = act_ref[...].reshape(E, A, NA)
    w4 = w3[..., None]
    z4 = w4 * act3[:, None, :, :] + (1.0 - w4) * pi3[:, None, :, :]
    ssum = jnp.sum(z4, axis=2)
    zz4 = (pi3[:, None, :, :] + ssum[:, :, None, :] - z4) * 0.125

    obs4 = jnp.broadcast_to(obs_ref[...].reshape(E, 1, A, D), (E, A, A, D))
    out_ref[...] = jnp.concatenate(
        [obs4.reshape(R * A, D), zz4.reshape(R * A, NA)], axis=-1).reshape(R, A, D + NA)
    w_ref[...] = w3.reshape(R, A, 1)


def kernel(h, policies, actions, obs_proc, edge_index,
           kW1, kb1, kW2, kb2, qW1, qb1, qW2, qb2):
    N, D = h.shape
    NA = policies.shape[1]
    OUT = kW2.shape[1]
    HID = kW1.shape[1]
    E = 32
    R = E * A
    grid = N // R

    kb1r = kb1.reshape(1, HID); kb2r = kb2.reshape(1, OUT)
    qb1r = qb1.reshape(1, HID); qb2r = qb2.reshape(1, OUT)

    node_spec = lambda w: pl.BlockSpec((R, w), lambda i: (i, 0))
    full_spec = lambda a, b: pl.BlockSpec((a, b), lambda i: (0, 0))

    out_final, w_final = pl.pallas_call(
        _body,
        grid=(grid,),
        in_specs=[
            node_spec(D), node_spec(NA), node_spec(NA), node_spec(D),
            full_spec(D, HID), full_spec(1, HID),
            full_spec(HID, OUT), full_spec(1, OUT),
            full_spec(D, HID), full_spec(1, HID),
            full_spec(HID, OUT), full_spec(1, OUT),
        ],
        out_specs=[
            pl.BlockSpec((R, A, D + NA), lambda i: (i, 0, 0)),
            pl.BlockSpec((R, A, 1), lambda i: (i, 0, 0)),
        ],
        out_shape=[
            jax.ShapeDtypeStruct((N, A, D + NA), jnp.float32),
            jax.ShapeDtypeStruct((N, A, 1), jnp.float32),
        ],
        compiler_params=pltpu.CompilerParams(
            dimension_semantics=("arbitrary",)),
    )(h, policies, actions, obs_proc,
      kW1, kb1r, kW2, kb2r, qW1, qb1r, qW2, qb2r)

    return out_final, w_final


# speedup vs baseline: 1.1627x; 1.1627x over previous
import jax
import jax.numpy as jnp
from jax.experimental import pallas as pl
from jax.experimental.pallas import tpu as pltpu

A = 8


def _body(h_ref, pi_ref, act_ref, obs_ref,
          kW1_ref, kb1_ref, kW2_ref, kb2_ref,
          qW1_ref, qb1_ref, qW2_ref, qb2_ref,
          out_ref, w_ref):
    R, D = h_ref.shape
    NA = pi_ref.shape[1]
    E = R // A

    hb = h_ref[...]
    K = jnp.dot(jnp.tanh(jnp.dot(hb, kW1_ref[...], preferred_element_type=jnp.float32)
                         + kb1_ref[...]),
                kW2_ref[...], preferred_element_type=jnp.float32) + kb2_ref[...]
    Q = jnp.dot(jnp.tanh(jnp.dot(hb, qW1_ref[...], preferred_element_type=jnp.float32)
                         + qb1_ref[...]),
                qW2_ref[...], preferred_element_type=jnp.float32) + qb2_ref[...]

    K3 = K.reshape(E, A, K.shape[-1])
    Q3 = Q.reshape(E, A, K.shape[-1])
    S = jnp.sum(Q3[:, :, None, :] * K3[:, None, :, :], axis=-1)
    w3 = jax.nn.sigmoid(S * 0.125)

    pi3 = pi_ref[...].reshape(E, A, NA)
    act3 = act_ref[...].reshape(E, A, NA)
    w4 = w3[..., None]
    z4 = w4 * act3[:, None, :, :] + (1.0 - w4) * pi3[:, None, :, :]
    ssum = jnp.sum(z4, axis=2)
    zz4 = (pi3[:, None, :, :] + ssum[:, :, None, :] - z4) * 0.125

    obs4 = jnp.broadcast_to(obs_ref[...].reshape(E, 1, A, D), (E, A, A, D))
    out_ref[...] = jnp.concatenate(
        [obs4.reshape(R * A, D), zz4.reshape(R * A, NA)], axis=-1).reshape(R, A, D + NA)
    w_ref[...] = w3.reshape(R, A)


def kernel(h, policies, actions, obs_proc, edge_index,
           kW1, kb1, kW2, kb2, qW1, qb1, qW2, qb2):
    N, D = h.shape
    NA = policies.shape[1]
    OUT = kW2.shape[1]
    HID = kW1.shape[1]
    E = 32
    R = E * A
    grid = N // R

    kb1r = kb1.reshape(1, HID); kb2r = kb2.reshape(1, OUT)
    qb1r = qb1.reshape(1, HID); qb2r = qb2.reshape(1, OUT)

    node_spec = lambda w: pl.BlockSpec((R, w), lambda i: (i, 0))
    full_spec = lambda a, b: pl.BlockSpec((a, b), lambda i: (0, 0))

    out_final, w_final = pl.pallas_call(
        _body,
        grid=(grid,),
        in_specs=[
            node_spec(D), node_spec(NA), node_spec(NA), node_spec(D),
            full_spec(D, HID), full_spec(1, HID),
            full_spec(HID, OUT), full_spec(1, OUT),
            full_spec(D, HID), full_spec(1, HID),
            full_spec(HID, OUT), full_spec(1, OUT),
        ],
        out_specs=[
            pl.BlockSpec((R, A, D + NA), lambda i: (i, 0, 0)),
            pl.BlockSpec((R, A), lambda i: (i, 0)),
        ],
        out_shape=[
            jax.ShapeDtypeStruct((N, A, D + NA), jnp.float32),
            jax.ShapeDtypeStruct((N, A), jnp.float32),
        ],
        compiler_params=pltpu.CompilerParams(
            dimension_semantics=("arbitrary",)),
    )(h, policies, actions, obs_proc,
      kW1, kb1r, kW2, kb2r, qW1, qb1r, qW2, qb2r)

    return out_final, w_final.reshape(N, A, 1)


# speedup vs baseline: 1.3398x; 1.1524x over previous
import jax
import jax.numpy as jnp
from jax.experimental import pallas as pl
from jax.experimental.pallas import tpu as pltpu

A = 8


def _body(h_ref, pi_ref, act_ref, obs_ref,
          kW1_ref, kb1_ref, kW2_ref, kb2_ref,
          qW1_ref, qb1_ref, qW2_ref, qb2_ref,
          out_ref, w_ref):
    R, D = h_ref.shape
    NA = pi_ref.shape[1]
    E = R // A

    hb = h_ref[...]
    K = jnp.dot(jnp.tanh(jnp.dot(hb, kW1_ref[...], preferred_element_type=jnp.float32)
                         + kb1_ref[...]),
                kW2_ref[...], preferred_element_type=jnp.float32) + kb2_ref[...]
    Q = jnp.dot(jnp.tanh(jnp.dot(hb, qW1_ref[...], preferred_element_type=jnp.float32)
                         + qb1_ref[...]),
                qW2_ref[...], preferred_element_type=jnp.float32) + qb2_ref[...]

    K3 = K.reshape(E, A, K.shape[-1])
    Q3 = Q.reshape(E, A, K.shape[-1])
    S = jnp.sum(Q3[:, :, None, :] * K3[:, None, :, :], axis=-1)
    w3 = jax.nn.sigmoid(S * 0.125)

    pi3 = pi_ref[...].reshape(E, A, NA)
    act3 = act_ref[...].reshape(E, A, NA)
    w4 = w3[..., None]
    z4 = w4 * act3[:, None, :, :] + (1.0 - w4) * pi3[:, None, :, :]
    ssum = jnp.sum(z4, axis=2)
    zz4 = (pi3[:, None, :, :] + ssum[:, :, None, :] - z4) * 0.125

    obs4 = jnp.broadcast_to(obs_ref[...].reshape(E, 1, A, D), (E, A, A, D))
    out_ref[:, :, 0:D] = obs4.reshape(R, A, D)
    out_ref[:, :, D:D + NA] = zz4.reshape(R, A, NA)
    w_ref[...] = w3.reshape(R, A)


def kernel(h, policies, actions, obs_proc, edge_index,
           kW1, kb1, kW2, kb2, qW1, qb1, qW2, qb2):
    N, D = h.shape
    NA = policies.shape[1]
    OUT = kW2.shape[1]
    HID = kW1.shape[1]
    E = 128
    R = E * A
    grid = N // R

    kb1r = kb1.reshape(1, HID); kb2r = kb2.reshape(1, OUT)
    qb1r = qb1.reshape(1, HID); qb2r = qb2.reshape(1, OUT)

    node_spec = lambda w: pl.BlockSpec((R, w), lambda i: (i, 0))
    full_spec = lambda a, b: pl.BlockSpec((a, b), lambda i: (0, 0))

    out_final, w_final = pl.pallas_call(
        _body,
        grid=(grid,),
        in_specs=[
            node_spec(D), node_spec(NA), node_spec(NA), node_spec(D),
            full_spec(D, HID), full_spec(1, HID),
            full_spec(HID, OUT), full_spec(1, OUT),
            full_spec(D, HID), full_spec(1, HID),
            full_spec(HID, OUT), full_spec(1, OUT),
        ],
        out_specs=[
            pl.BlockSpec((R, A, D + NA), lambda i: (i, 0, 0)),
            pl.BlockSpec((R, A), lambda i: (i, 0)),
        ],
        out_shape=[
            jax.ShapeDtypeStruct((N, A, D + NA), jnp.float32),
            jax.ShapeDtypeStruct((N, A), jnp.float32),
        ],
        compiler_params=pltpu.CompilerParams(
            dimension_semantics=("arbitrary",)),
    )(h, policies, actions, obs_proc,
      kW1, kb1r, kW2, kb2r, qW1, qb1r, qW2, qb2r)

    return out_final, w_final.reshape(N, A, 1)
